# Initial kernel scaffold; baseline (speedup 1.0000x reference)
import jax
import jax.numpy as jnp
from jax.experimental import pallas as pl


def kernel(cur_proba, proba, outs, is_ended):
    raise NotImplementedError("write your pallas kernel here")


# speedup vs baseline: 1.1767x; 1.1767x over previous
import jax
import jax.numpy as jnp
from jax.experimental import pallas as pl

END_TOKEN = 2
NEG_INF = float("-inf")


def _tc_body(cp_ref, pr_ref, ie_ref, outs_ref,
             vals_ref, voc_ref, beam_ref, ended_ref, outs_new_ref):
    E, V = cp_ref.shape[1], cp_ref.shape[2]
    cp = cp_ref[0]
    pr = pr_ref[0, 0]
    ended = ie_ref[0, 0]

    pr_b = jax.lax.broadcast_in_dim(pr, (E, V), (0,))
    ended_col = jax.lax.broadcast_in_dim(ended, (E, V), (0,)) > 0
    col_is_end = jax.lax.broadcasted_iota(jnp.int32, (E, V), 1) == END_TOKEN
    base = pr_b + cp
    p = jnp.where(ended_col, jnp.where(col_is_end, pr_b, NEG_INF), base)

    g_iota = (jax.lax.broadcasted_iota(jnp.int32, (E, V), 0) * V
              + jax.lax.broadcasted_iota(jnp.int32, (E, V), 1))
    BIG = jnp.int32(2**30)

    vals = []
    gms = []
    y = p
    for _ in range(E):
        m = jnp.max(y)
        eq = y == m
        gm = jnp.min(jnp.where(eq, g_iota, BIG))
        vals.append(m)
        gms.append(gm)
        y = jnp.where(g_iota == gm, NEG_INF, y)

    vals_v = jnp.stack(vals)
    gm_v = jnp.stack(gms)
    voc = gm_v & jnp.int32(V - 1)
    beam = gm_v >> 15

    vals_ref[0, 0, :] = vals_v
    voc_ref[0, 0, :] = voc
    beam_ref[0, 0, :] = beam

    eg = jnp.zeros((E,), jnp.int32)
    for e in range(E):
        eg = jnp.where(beam == e, ended[e], eg)
    ended_ref[0, 0, :] = jnp.where(voc == END_TOKEN, 1, eg)

    L = outs_ref.shape[1]
    acc = jnp.zeros((L, E), jnp.int32)
    beam_row = jax.lax.broadcast_in_dim(beam, (L, E), (1,))
    for e in range(E):
        col = jax.lax.broadcast_in_dim(outs_ref[0, :, e], (L, E), (0,))
        acc = jnp.where(beam_row == e, col, acc)
    outs_new_ref[0, :L, :] = acc
    outs_new_ref[0, pl.ds(L, 1), :] = jax.lax.broadcast_in_dim(voc, (1, E), (1,))


def kernel(cur_proba, proba, outs, is_ended):
    _, _, V = cur_proba.shape
    L, B, E = outs.shape
    cp = cur_proba.reshape(B, E, V)
    pr = proba.reshape(B, 1, E)
    ie = is_ended.astype(jnp.int32).reshape(B, 1, E)
    outs_t = outs.transpose(1, 0, 2)

    out_shapes = (
        jax.ShapeDtypeStruct((B, 1, E), jnp.float32),
        jax.ShapeDtypeStruct((B, 1, E), jnp.int32),
        jax.ShapeDtypeStruct((B, 1, E), jnp.int32),
        jax.ShapeDtypeStruct((B, 1, E), jnp.int32),
        jax.ShapeDtypeStruct((B, L + 1, E), jnp.int32),
    )
    grid = (B,)
    small = lambda: pl.BlockSpec((1, 1, E), lambda b: (b, 0, 0))
    vals, voc, beam, ended, outs_new_t = pl.pallas_call(
        _tc_body,
        grid=grid,
        in_specs=[
            pl.BlockSpec((1, E, V), lambda b: (b, 0, 0)),
            small(),
            small(),
            pl.BlockSpec((1, L, E), lambda b: (b, 0, 0)),
        ],
        out_specs=[
            small(), small(), small(), small(),
            pl.BlockSpec((1, L + 1, E), lambda b: (b, 0, 0)),
        ],
        out_shape=out_shapes,
    )(cp, pr, ie, outs_t)

    cur_input = voc.reshape(B * E, 1)
    proba_new = vals.reshape(B, E)
    outs_new = outs_new_t.transpose(1, 0, 2)
    is_ended_new = ended.reshape(B, E).astype(bool)
    topk_beam = beam.reshape(B, E)
    return (cur_input, proba_new, outs_new, is_ended_new, topk_beam)


# speedup vs baseline: 2.0009x; 1.7005x over previous
import functools
import jax
import jax.numpy as jnp
from jax import lax
from jax.experimental import pallas as pl
from jax.experimental.pallas import tpu as pltpu
from jax.experimental.pallas import tpu_sc as plsc

END_TOKEN = 2
NEG_INF = float("-inf")
POS_INF = float("inf")
BB, EE, VV, LL = 32, 8, 32768, 128
CHUNK = 16384
GRP = 128
NLANE = 16


def _spl_f(x):
    return lax.broadcast_in_dim(x, (NLANE,), ())


def _spl_i(x):
    return lax.broadcast_in_dim(x, (NLANE,), ())


def _iota():
    return lax.broadcasted_iota(jnp.int32, (NLANE,), 0)


def _dg(x, i):
    dn = lax.GatherDimensionNumbers(offset_dims=(), collapsed_slice_dims=(0,),
                                    start_index_map=(0,))
    return lax.gather(x, lax.broadcast_in_dim(i, (NLANE, 1), (0,)), dn,
                      slice_sizes=(1,), mode=lax.GatherScatterMode.PROMISE_IN_BOUNDS)


def _bubble(cands, v, g):
    vals = list(cands[:8])
    idxs = list(cands[8:])
    for j in range(8):
        gt = v > vals[j]
        nv = jnp.where(gt, v, vals[j])
        ng = jnp.where(gt, g, idxs[j])
        v = jnp.where(gt, vals[j], v)
        g = jnp.where(gt, idxs[j], g)
        vals[j], idxs[j] = nv, ng
    return tuple(vals) + tuple(idxs)


def _sc_entry(cp_hbm, pr_hbm, ie_hbm, outs_hbm,
              vals_hbm, voc_hbm, beam_hbm, ended_hbm, outsnew_hbm,
              buf, pv, iev, lmv, outs_v, outnew_v, smf, smi):
    ncores = 2
    b = lax.axis_index("s") * ncores + lax.axis_index("c")
    row0 = b * (EE * VV)

    pv[...] = jnp.zeros((NLANE,), jnp.float32)
    iev[...] = jnp.zeros((NLANE,), jnp.int32)
    pltpu.sync_copy(pr_hbm.at[pl.ds(b * EE, EE)], pv.at[pl.ds(0, EE)])
    pltpu.sync_copy(ie_hbm.at[pl.ds(b * EE, EE)], iev.at[pl.ds(0, EE)])
    pltpu.sync_copy(outs_hbm.at[pl.ds(b * (LL * EE), LL * EE)], outs_v)
    pvv = pv[...]
    ievv = iev[...]

    iot = _iota()

    def row_a(e, _):
        def chunk_a(c, accs):
            off = pl.multiple_of(row0 + e * VV + c * CHUNK, CHUNK)
            pltpu.sync_copy(cp_hbm.at[pl.ds(off, CHUNK)], buf)

            def grp_a(i, accs):
                base = i * GRP
                return tuple(
                    jnp.maximum(accs[k], buf[pl.ds(base + k * NLANE, NLANE)])
                    for k in range(8))

            return lax.fori_loop(0, CHUNK // GRP, grp_a, accs)

        accs = tuple(jnp.full((NLANE,), NEG_INF, jnp.float32) for _ in range(8))
        accs = lax.fori_loop(0, VV // CHUNK, chunk_a, accs)
        m01 = jnp.maximum(jnp.maximum(accs[0], accs[1]),
                          jnp.maximum(accs[2], accs[3]))
        m23 = jnp.maximum(jnp.maximum(accs[4], accs[5]),
                          jnp.maximum(accs[6], accs[7]))
        lmv[pl.ds(e * NLANE, NLANE)] = jnp.maximum(m01, m23)
        return 0

    lax.fori_loop(0, EE, row_a, 0)

    cvs = []
    for e in range(8):
        idx_e = _spl_i(jnp.int32(e))
        p_spl = _dg(pvv, idx_e)
        end_spl = _dg(ievv, idx_e) > 0
        lane0 = iot == 0
        single = jnp.where(lane0, p_spl, NEG_INF)
        cvs.append(jnp.where(end_spl, single,
                             lmv[pl.ds(e * NLANE, NLANE)] + p_spl))
    tval = None
    for _ in range(8):
        m01 = jnp.maximum(jnp.maximum(cvs[0], cvs[1]),
                          jnp.maximum(cvs[2], cvs[3]))
        m23 = jnp.maximum(jnp.maximum(cvs[4], cvs[5]),
                          jnp.maximum(cvs[6], cvs[7]))
        mscal = lax.reduce_max(jnp.maximum(m01, m23), (0,))
        mspl = _spl_f(mscal)
        cvs = [jnp.where(cv == mspl, NEG_INF, cv) for cv in cvs]
        tval = mspl

    lane0 = iot == 0
    lt8 = iot < 8
    endvec = jnp.where((iev[...] > 0) & lt8, pv[...], NEG_INF)
    endidx = jnp.where(lt8, iot * VV + END_TOKEN, 0)
    cands = ((endvec,) + tuple(jnp.full((NLANE,), NEG_INF, jnp.float32)
                               for _ in range(7))
             + (endidx,) + tuple(jnp.zeros((NLANE,), jnp.int32)
                                 for _ in range(7)))

    def row_b(e, cands):
        idx_e = _spl_i(e)
        p_spl = _dg(pvv, idx_e)
        end_spl = _dg(ievv, idx_e) > 0
        trow = jnp.where(end_spl, POS_INF, tval)
        gb = _spl_i(e * VV) + iot

        def chunk_b(c, cands):
            off = pl.multiple_of(row0 + e * VV + c * CHUNK, CHUNK)
            pltpu.sync_copy(cp_hbm.at[pl.ds(off, CHUNK)], buf)
            gchunk = gb + _spl_i(c * CHUNK)

            def grp_b(i, cands):
                base = i * GRP
                xs = [buf[pl.ds(base + k * NLANE, NLANE)] + p_spl
                      for k in range(8)]
                hs = [x >= trow for x in xs]
                h01 = (hs[0] | hs[1]) | (hs[2] | hs[3])
                h23 = (hs[4] | hs[5]) | (hs[6] | hs[7])
                hit = jnp.any(h01 | h23)

                def slow(cands):
                    gk = gchunk + _spl_i(base)
                    for k in range(8):
                        cands = _bubble(cands, xs[k],
                                        gk + _spl_i(k * NLANE))
                    return cands

                return lax.cond(hit, slow, lambda c: c, cands)

            return lax.fori_loop(0, CHUNK // GRP, grp_b, cands)

        return lax.fori_loop(0, VV // CHUNK, chunk_b, cands)

    cands = lax.fori_loop(0, EE, row_b, cands)
    cvals = list(cands[:8])
    cidxs = list(cands[8:])

    BIG = jnp.int32(2**30)
    outv = jnp.zeros((NLANE,), jnp.float32)
    outg = jnp.zeros((NLANE,), jnp.int32)
    for k in range(8):
        m01 = jnp.maximum(jnp.maximum(cvals[0], cvals[1]),
                          jnp.maximum(cvals[2], cvals[3]))
        m23 = jnp.maximum(jnp.maximum(cvals[4], cvals[5]),
                          jnp.maximum(cvals[6], cvals[7]))
        mspl = _spl_f(lax.reduce_max(jnp.maximum(m01, m23), (0,)))
        eqs = [cv == mspl for cv in cvals]
        gs = [jnp.where(eqs[j], cidxs[j], BIG) for j in range(8)]
        g01 = jnp.minimum(jnp.minimum(gs[0], gs[1]), jnp.minimum(gs[2], gs[3]))
        g23 = jnp.minimum(jnp.minimum(gs[4], gs[5]), jnp.minimum(gs[6], gs[7]))
        gspl = _spl_i(lax.reduce_min(jnp.minimum(g01, g23), (0,)))
        sel = iot == k
        outv = jnp.where(sel, mspl, outv)
        outg = jnp.where(sel, gspl, outg)
        for j in range(8):
            rm = eqs[j] & (cidxs[j] == gspl)
            cvals[j] = jnp.where(rm, NEG_INF, cvals[j])

    voc = outg & jnp.int32(VV - 1)
    beam = lax.shift_right_logical(outg, 15)
    ended_g = _dg(ievv, beam)
    newend = jnp.where(voc == END_TOKEN, 1, ended_g)

    smf[...] = outv
    pltpu.sync_copy(smf.at[pl.ds(0, EE)], vals_hbm.at[pl.ds(b * EE, EE)])
    smi[...] = voc
    pltpu.sync_copy(smi.at[pl.ds(0, EE)], voc_hbm.at[pl.ds(b * EE, EE)])
    smi[...] = newend
    pltpu.sync_copy(smi.at[pl.ds(0, EE)], ended_hbm.at[pl.ds(b * EE, EE)])
    smi[...] = beam
    pltpu.sync_copy(smi.at[pl.ds(0, EE)], beam_hbm.at[pl.ds(b * EE, EE)])

    beam2 = _dg(beam, iot & 7)
    off0 = beam2 + jnp.where(iot >= 8, jnp.int32(EE), jnp.int32(0))

    def lp(i, _):
        x = outs_v[pl.ds(i * NLANE, NLANE)]
        outnew_v[pl.ds(i * NLANE, NLANE)] = _dg(x, off0)
        return 0

    lax.fori_loop(0, (LL * EE) // NLANE, lp, 0)
    outnew_v[pl.ds(LL * EE, NLANE)] = voc
    pltpu.sync_copy(outnew_v.at[pl.ds(0, (LL + 1) * EE)],
                    outsnew_hbm.at[pl.ds(b * (LL + 1) * EE, (LL + 1) * EE)])


def kernel(cur_proba, proba, outs, is_ended):
    _, _, V = cur_proba.shape
    L, B, E = outs.shape
    cp = cur_proba.reshape(-1)
    pr = proba.reshape(-1)
    ie = is_ended.astype(jnp.int32).reshape(-1)
    outs_t = outs.transpose(1, 0, 2).reshape(-1)

    mesh = plsc.VectorSubcoreMesh(core_axis_name="c", subcore_axis_name="s",
                                  num_cores=2, num_subcores=16)
    run = functools.partial(
        pl.kernel,
        mesh=mesh,
        compiler_params=pltpu.CompilerParams(needs_layout_passes=False),
        out_type=(
            jax.ShapeDtypeStruct((B * E,), jnp.float32),
            jax.ShapeDtypeStruct((B * E,), jnp.int32),
            jax.ShapeDtypeStruct((B * E,), jnp.int32),
            jax.ShapeDtypeStruct((B * E,), jnp.int32),
            jax.ShapeDtypeStruct((B * (L + 1) * E,), jnp.int32),
        ),
        scratch_types=[
            pltpu.VMEM((CHUNK,), jnp.float32),
            pltpu.VMEM((NLANE,), jnp.float32),
            pltpu.VMEM((NLANE,), jnp.int32),
            pltpu.VMEM((E * NLANE,), jnp.float32),
            pltpu.VMEM((L * E,), jnp.int32),
            pltpu.VMEM(((L + 1) * E + 8,), jnp.int32),
            pltpu.VMEM((NLANE,), jnp.float32),
            pltpu.VMEM((NLANE,), jnp.int32),
        ],
    )(_sc_entry)

    vals, voc, beam, ended, outsnew = run(cp, pr, ie, outs_t)

    cur_input = voc.reshape(B * E, 1)
    proba_new = vals.reshape(B, E)
    outs_new = outsnew.reshape(B, L + 1, E).transpose(1, 0, 2)
    is_ended_new = ended.reshape(B, E).astype(bool)
    topk_beam = beam.reshape(B, E)
    return (cur_input, proba_new, outs_new, is_ended_new, topk_beam)


# speedup vs baseline: 2.5597x; 1.2793x over previous
import functools
import jax
import jax.numpy as jnp
from jax import lax
from jax.experimental import pallas as pl
from jax.experimental.pallas import tpu as pltpu
from jax.experimental.pallas import tpu_sc as plsc

END_TOKEN = 2
NEG_INF = float("-inf")
POS_INF = float("inf")
BB, EE, VV, LL = 32, 8, 32768, 128
CHUNK = 16384
GRP = 128
NLANE = 16


def _spl_f(x):
    return lax.broadcast_in_dim(x, (NLANE,), ())


def _spl_i(x):
    return lax.broadcast_in_dim(x, (NLANE,), ())


def _iota():
    return lax.broadcasted_iota(jnp.int32, (NLANE,), 0)


def _dg(x, i):
    dn = lax.GatherDimensionNumbers(offset_dims=(), collapsed_slice_dims=(0,),
                                    start_index_map=(0,))
    return lax.gather(x, lax.broadcast_in_dim(i, (NLANE, 1), (0,)), dn,
                      slice_sizes=(1,), mode=lax.GatherScatterMode.PROMISE_IN_BOUNDS)


def _bubble(cands, v, g):
    vals = list(cands[:8])
    idxs = list(cands[8:])
    for j in range(8):
        gt = v > vals[j]
        nv = jnp.where(gt, v, vals[j])
        ng = jnp.where(gt, g, idxs[j])
        v = jnp.where(gt, vals[j], v)
        g = jnp.where(gt, idxs[j], g)
        vals[j], idxs[j] = nv, ng
    return tuple(vals) + tuple(idxs)


def _sc_entry(cp_hbm, pr_hbm, ie_hbm, outs_hbm,
              vals_hbm, voc_hbm, beam_hbm, ended_hbm, outsnew_hbm,
              buf0, buf1, pv, iev, lmv, outs_v, outnew_v, smf, smi,
              sem0, sem1):
    ncores = 2
    b = lax.axis_index("s") * ncores + lax.axis_index("c")
    row0 = b * (EE * VV)
    nchunk = (EE * VV) // CHUNK

    def _start(off, bufx, semx):
        pltpu.async_copy(cp_hbm.at[pl.ds(pl.multiple_of(off, 8), CHUNK)],
                         bufx, semx)

    def _wait(bufx, semx):
        pltpu.make_async_copy(cp_hbm.at[pl.ds(0, CHUNK)], bufx, semx).wait()

    def _nxt(g):
        return jnp.where(g + 1 >= nchunk, row0, row0 + (g + 1) * CHUNK)

    pv[...] = jnp.zeros((NLANE,), jnp.float32)
    iev[...] = jnp.zeros((NLANE,), jnp.int32)
    pltpu.sync_copy(pr_hbm.at[pl.ds(b * EE, EE)], pv.at[pl.ds(0, EE)])
    pltpu.sync_copy(ie_hbm.at[pl.ds(b * EE, EE)], iev.at[pl.ds(0, EE)])
    pltpu.sync_copy(outs_hbm.at[pl.ds(b * (LL * EE), LL * EE)], outs_v)
    pvv = pv[...]
    ievv = iev[...]

    iot = _iota()

    _start(row0, buf0, sem0)

    def row_a(e, _):
        def half_a(c, bufc, semc, bufn, semn, accs):
            g = e * (VV // CHUNK) + c
            _start(_nxt(g), bufn, semn)
            _wait(bufc, semc)

            def grp_a(i, accs):
                base = i * GRP
                return tuple(
                    jnp.maximum(accs[k], bufc[pl.ds(base + k * NLANE, NLANE)])
                    for k in range(8))

            return lax.fori_loop(0, CHUNK // GRP, grp_a, accs)

        accs = tuple(jnp.full((NLANE,), NEG_INF, jnp.float32) for _ in range(8))
        accs = half_a(0, buf0, sem0, buf1, sem1, accs)
        accs = half_a(1, buf1, sem1, buf0, sem0, accs)
        m01 = jnp.maximum(jnp.maximum(accs[0], accs[1]),
                          jnp.maximum(accs[2], accs[3]))
        m23 = jnp.maximum(jnp.maximum(accs[4], accs[5]),
                          jnp.maximum(accs[6], accs[7]))
        lmv[pl.ds(e * NLANE, NLANE)] = jnp.maximum(m01, m23)
        return 0

    lax.fori_loop(0, EE, row_a, 0)

    cvs = []
    for e in range(8):
        idx_e = _spl_i(jnp.int32(e))
        p_spl = _dg(pvv, idx_e)
        end_spl = _dg(ievv, idx_e) > 0
        lane0 = iot == 0
        single = jnp.where(lane0, p_spl, NEG_INF)
        cvs.append(jnp.where(end_spl, single,
                             lmv[pl.ds(e * NLANE, NLANE)] + p_spl))
    tval = None
    for _ in range(8):
        m01 = jnp.maximum(jnp.maximum(cvs[0], cvs[1]),
                          jnp.maximum(cvs[2], cvs[3]))
        m23 = jnp.maximum(jnp.maximum(cvs[4], cvs[5]),
                          jnp.maximum(cvs[6], cvs[7]))
        mscal = lax.reduce_max(jnp.maximum(m01, m23), (0,))
        mspl = _spl_f(mscal)
        cvs = [jnp.where(cv == mspl, NEG_INF, cv) for cv in cvs]
        tval = mspl

    lane0 = iot == 0
    lt8 = iot < 8
    endvec = jnp.where((iev[...] > 0) & lt8, pv[...], NEG_INF)
    endidx = jnp.where(lt8, iot * VV + END_TOKEN, 0)
    cands = ((endvec,) + tuple(jnp.full((NLANE,), NEG_INF, jnp.float32)
                               for _ in range(7))
             + (endidx,) + tuple(jnp.zeros((NLANE,), jnp.int32)
                                 for _ in range(7)))

    def row_b(e, cands):
        idx_e = _spl_i(e)
        p_spl = _dg(pvv, idx_e)
        end_spl = _dg(ievv, idx_e) > 0
        trow = jnp.where(end_spl, POS_INF, tval)
        gb = _spl_i(e * VV) + iot

        def chunk_b(c, bufc, semc, bufn, semn, cands):
            g = e * (VV // CHUNK) + c
            _start(_nxt(g), bufn, semn)
            _wait(bufc, semc)
            gchunk = gb + _spl_i(c * CHUNK)

            def grp_b(i, cands):
                base = i * GRP
                xs = [bufc[pl.ds(base + k * NLANE, NLANE)] + p_spl
                      for k in range(8)]
                hs = [x >= trow for x in xs]
                h01 = (hs[0] | hs[1]) | (hs[2] | hs[3])
                h23 = (hs[4] | hs[5]) | (hs[6] | hs[7])
                hit = jnp.any(h01 | h23)

                def slow(cands):
                    gk = gchunk + _spl_i(base)
                    for k in range(8):
                        cands = _bubble(cands, xs[k],
                                        gk + _spl_i(k * NLANE))
                    return cands

                return lax.cond(hit, slow, lambda c: c, cands)

            return lax.fori_loop(0, CHUNK // GRP, grp_b, cands)

        cands = chunk_b(0, buf0, sem0, buf1, sem1, cands)
        return chunk_b(1, buf1, sem1, buf0, sem0, cands)

    cands = lax.fori_loop(0, EE, row_b, cands)
    _wait(buf0, sem0)
    cvals = list(cands[:8])
    cidxs = list(cands[8:])

    BIG = jnp.int32(2**30)
    outv = jnp.zeros((NLANE,), jnp.float32)
    outg = jnp.zeros((NLANE,), jnp.int32)
    for k in range(8):
        m01 = jnp.maximum(jnp.maximum(cvals[0], cvals[1]),
                          jnp.maximum(cvals[2], cvals[3]))
        m23 = jnp.maximum(jnp.maximum(cvals[4], cvals[5]),
                          jnp.maximum(cvals[6], cvals[7]))
        mspl = _spl_f(lax.reduce_max(jnp.maximum(m01, m23), (0,)))
        eqs = [cv == mspl for cv in cvals]
        gs = [jnp.where(eqs[j], cidxs[j], BIG) for j in range(8)]
        g01 = jnp.minimum(jnp.minimum(gs[0], gs[1]), jnp.minimum(gs[2], gs[3]))
        g23 = jnp.minimum(jnp.minimum(gs[4], gs[5]), jnp.minimum(gs[6], gs[7]))
        gspl = _spl_i(lax.reduce_min(jnp.minimum(g01, g23), (0,)))
        sel = iot == k
        outv = jnp.where(sel, mspl, outv)
        outg = jnp.where(sel, gspl, outg)
        for j in range(8):
            rm = eqs[j] & (cidxs[j] == gspl)
            cvals[j] = jnp.where(rm, NEG_INF, cvals[j])

    voc = outg & jnp.int32(VV - 1)
    beam = lax.shift_right_logical(outg, 15)
    ended_g = _dg(ievv, beam)
    newend = jnp.where(voc == END_TOKEN, 1, ended_g)

    smf[...] = outv
    pltpu.sync_copy(smf.at[pl.ds(0, EE)], vals_hbm.at[pl.ds(b * EE, EE)])
    smi[...] = voc
    pltpu.sync_copy(smi.at[pl.ds(0, EE)], voc_hbm.at[pl.ds(b * EE, EE)])
    smi[...] = newend
    pltpu.sync_copy(smi.at[pl.ds(0, EE)], ended_hbm.at[pl.ds(b * EE, EE)])
    smi[...] = beam
    pltpu.sync_copy(smi.at[pl.ds(0, EE)], beam_hbm.at[pl.ds(b * EE, EE)])

    beam2 = _dg(beam, iot & 7)
    off0 = beam2 + jnp.where(iot >= 8, jnp.int32(EE), jnp.int32(0))

    def lp(i, _):
        x = outs_v[pl.ds(i * NLANE, NLANE)]
        outnew_v[pl.ds(i * NLANE, NLANE)] = _dg(x, off0)
        return 0

    lax.fori_loop(0, (LL * EE) // NLANE, lp, 0)
    outnew_v[pl.ds(LL * EE, NLANE)] = voc
    pltpu.sync_copy(outnew_v.at[pl.ds(0, (LL + 1) * EE)],
                    outsnew_hbm.at[pl.ds(b * (LL + 1) * EE, (LL + 1) * EE)])


def kernel(cur_proba, proba, outs, is_ended):
    _, _, V = cur_proba.shape
    L, B, E = outs.shape
    cp = cur_proba.reshape(-1)
    pr = proba.reshape(-1)
    ie = is_ended.astype(jnp.int32).reshape(-1)
    outs_t = outs.transpose(1, 0, 2).reshape(-1)

    mesh = plsc.VectorSubcoreMesh(core_axis_name="c", subcore_axis_name="s",
                                  num_cores=2, num_subcores=16)
    run = functools.partial(
        pl.kernel,
        mesh=mesh,
        compiler_params=pltpu.CompilerParams(needs_layout_passes=False),
        out_type=(
            jax.ShapeDtypeStruct((B * E,), jnp.float32),
            jax.ShapeDtypeStruct((B * E,), jnp.int32),
            jax.ShapeDtypeStruct((B * E,), jnp.int32),
            jax.ShapeDtypeStruct((B * E,), jnp.int32),
            jax.ShapeDtypeStruct((B * (L + 1) * E,), jnp.int32),
        ),
        scratch_types=[
            pltpu.VMEM((CHUNK,), jnp.float32),
            pltpu.VMEM((CHUNK,), jnp.float32),
            pltpu.VMEM((NLANE,), jnp.float32),
            pltpu.VMEM((NLANE,), jnp.int32),
            pltpu.VMEM((E * NLANE,), jnp.float32),
            pltpu.VMEM((L * E,), jnp.int32),
            pltpu.VMEM(((L + 1) * E + 8,), jnp.int32),
            pltpu.VMEM((NLANE,), jnp.float32),
            pltpu.VMEM((NLANE,), jnp.int32),
            pltpu.SemaphoreType.DMA,
            pltpu.SemaphoreType.DMA,
        ],
    )(_sc_entry)

    vals, voc, beam, ended, outsnew = run(cp, pr, ie, outs_t)

    cur_input = voc.reshape(B * E, 1)
    proba_new = vals.reshape(B, E)
    outs_new = outsnew.reshape(B, L + 1, E).transpose(1, 0, 2)
    is_ended_new = ended.reshape(B, E).astype(bool)
    topk_beam = beam.reshape(B, E)
    return (cur_input, proba_new, outs_new, is_ended_new, topk_beam)


# speedup vs baseline: 3.4386x; 1.3433x over previous
import functools
import jax
import jax.numpy as jnp
from jax import lax
from jax.experimental import pallas as pl
from jax.experimental.pallas import tpu as pltpu
from jax.experimental.pallas import tpu_sc as plsc

END_TOKEN = 2
NEG_INF = float("-inf")
POS_INF = float("inf")
BB, EE, VV, LL = 32, 8, 32768, 128
CHUNK = 16384
GRP = 256
NLANE = 16


def _spl_f(x):
    return lax.broadcast_in_dim(x, (NLANE,), ())


def _spl_i(x):
    return lax.broadcast_in_dim(x, (NLANE,), ())


def _iota():
    return lax.broadcasted_iota(jnp.int32, (NLANE,), 0)


def _dg(x, i):
    dn = lax.GatherDimensionNumbers(offset_dims=(), collapsed_slice_dims=(0,),
                                    start_index_map=(0,))
    return lax.gather(x, lax.broadcast_in_dim(i, (NLANE, 1), (0,)), dn,
                      slice_sizes=(1,), mode=lax.GatherScatterMode.PROMISE_IN_BOUNDS)


def _bubble(cands, v, g):
    vals = list(cands[:8])
    idxs = list(cands[8:])
    for j in range(8):
        gt = v > vals[j]
        nv = jnp.where(gt, v, vals[j])
        ng = jnp.where(gt, g, idxs[j])
        v = jnp.where(gt, vals[j], v)
        g = jnp.where(gt, idxs[j], g)
        vals[j], idxs[j] = nv, ng
    return tuple(vals) + tuple(idxs)


def _sc_entry(cp_hbm, pr_hbm, ie_hbm, outs_hbm,
              vals_hbm, voc_hbm, beam_hbm, ended_hbm, outsnew_hbm,
              buf0, buf1, pv, iev, lmv, outs_v, outnew_v, smf, smi,
              sem0, sem1, sem2):
    ncores = 2
    b = lax.axis_index("s") * ncores + lax.axis_index("c")
    row0 = b * (EE * VV)
    nchunk = (EE * VV) // CHUNK

    def _start(off, bufx, semx):
        pltpu.async_copy(cp_hbm.at[pl.ds(pl.multiple_of(off, 8), CHUNK)],
                         bufx, semx)

    def _wait(bufx, semx):
        pltpu.make_async_copy(cp_hbm.at[pl.ds(0, CHUNK)], bufx, semx).wait()

    def _nxt(g):
        return jnp.where(g + 1 >= nchunk, row0, row0 + (g + 1) * CHUNK)

    pv[...] = jnp.zeros((NLANE,), jnp.float32)
    iev[...] = jnp.zeros((NLANE,), jnp.int32)
    iot = _iota()

    _start(row0, buf0, sem0)
    cp_pr = pltpu.make_async_copy(pr_hbm.at[pl.ds(b * EE, EE)],
                                  pv.at[pl.ds(0, EE)], sem2)
    cp_ie = pltpu.make_async_copy(ie_hbm.at[pl.ds(b * EE, EE)],
                                  iev.at[pl.ds(0, EE)], sem2)
    cp_outs = pltpu.make_async_copy(
        outs_hbm.at[pl.ds(b * (LL * EE), LL * EE)], outs_v, sem2)
    cp_pr.start()
    cp_ie.start()
    cp_outs.start()

    def row_a(e, _):
        def half_a(c, bufc, semc, bufn, semn, accs):
            g = e * (VV // CHUNK) + c
            _start(_nxt(g), bufn, semn)
            _wait(bufc, semc)

            def grp_a(i, accs):
                base = i * GRP
                return tuple(
                    jnp.maximum(accs[k], bufc[pl.ds(base + k * NLANE, NLANE)])
                    for k in range(16))

            return lax.fori_loop(0, CHUNK // GRP, grp_a, accs)

        accs = tuple(jnp.full((NLANE,), NEG_INF, jnp.float32) for _ in range(16))
        accs = half_a(0, buf0, sem0, buf1, sem1, accs)
        accs = half_a(1, buf1, sem1, buf0, sem0, accs)
        m = list(accs)
        while len(m) > 1:
            m = [jnp.maximum(m[2 * i], m[2 * i + 1]) for i in range(len(m) // 2)]
        lmv[pl.ds(e * NLANE, NLANE)] = m[0]
        return 0

    lax.fori_loop(0, EE, row_a, 0)
    cp_pr.wait()
    cp_ie.wait()
    cp_outs.wait()
    pvv = pv[...]
    ievv = iev[...]

    cvs = []
    for e in range(8):
        idx_e = _spl_i(jnp.int32(e))
        p_spl = _dg(pvv, idx_e)
        end_spl = _dg(ievv, idx_e) > 0
        lane0 = iot == 0
        single = jnp.where(lane0, p_spl, NEG_INF)
        cvs.append(jnp.where(end_spl, single,
                             lmv[pl.ds(e * NLANE, NLANE)] + p_spl))
    tval = None
    for _ in range(8):
        m01 = jnp.maximum(jnp.maximum(cvs[0], cvs[1]),
                          jnp.maximum(cvs[2], cvs[3]))
        m23 = jnp.maximum(jnp.maximum(cvs[4], cvs[5]),
                          jnp.maximum(cvs[6], cvs[7]))
        mscal = lax.reduce_max(jnp.maximum(m01, m23), (0,))
        mspl = _spl_f(mscal)
        cvs = [jnp.where(cv == mspl, NEG_INF, cv) for cv in cvs]
        tval = mspl

    lane0 = iot == 0
    lt8 = iot < 8
    endvec = jnp.where((iev[...] > 0) & lt8, pv[...], NEG_INF)
    endidx = jnp.where(lt8, iot * VV + END_TOKEN, 0)
    cands = ((endvec,) + tuple(jnp.full((NLANE,), NEG_INF, jnp.float32)
                               for _ in range(7))
             + (endidx,) + tuple(jnp.zeros((NLANE,), jnp.int32)
                                 for _ in range(7)))

    def row_b(e, cands):
        idx_e = _spl_i(e)
        p_spl = _dg(pvv, idx_e)
        end_spl = _dg(ievv, idx_e) > 0
        margin = (jnp.abs(tval) + jnp.abs(p_spl) + 1.0) * 1e-5
        trow = jnp.where(end_spl, POS_INF, (tval - p_spl) - margin)
        gb = _spl_i(e * VV) + iot

        def chunk_b(c, bufc, semc, bufn, semn, cands):
            g = e * (VV // CHUNK) + c
            _start(_nxt(g), bufn, semn)
            _wait(bufc, semc)
            gchunk = gb + _spl_i(c * CHUNK)

            def grp_b(i, cands):
                base = i * GRP
                xs = [bufc[pl.ds(base + k * NLANE, NLANE)]
                      for k in range(16)]
                m = list(xs)
                while len(m) > 1:
                    m = [jnp.maximum(m[2 * i2], m[2 * i2 + 1])
                         for i2 in range(len(m) // 2)]
                hit = jnp.any(m[0] >= trow)

                def slow(cands):
                    gk = gchunk + _spl_i(base)
                    for k in range(16):
                        cands = _bubble(cands, xs[k] + p_spl,
                                        gk + _spl_i(k * NLANE))
                    return cands

                return lax.cond(hit, slow, lambda c: c, cands)

            return lax.fori_loop(0, CHUNK // GRP, grp_b, cands)

        cands = chunk_b(0, buf0, sem0, buf1, sem1, cands)
        return chunk_b(1, buf1, sem1, buf0, sem0, cands)

    cands = lax.fori_loop(0, EE, row_b, cands)
    _wait(buf0, sem0)
    cvals = list(cands[:8])
    cidxs = list(cands[8:])

    BIG = jnp.int32(2**30)
    outv = jnp.zeros((NLANE,), jnp.float32)
    outg = jnp.zeros((NLANE,), jnp.int32)
    for k in range(8):
        m01 = jnp.maximum(jnp.maximum(cvals[0], cvals[1]),
                          jnp.maximum(cvals[2], cvals[3]))
        m23 = jnp.maximum(jnp.maximum(cvals[4], cvals[5]),
                          jnp.maximum(cvals[6], cvals[7]))
        mspl = _spl_f(lax.reduce_max(jnp.maximum(m01, m23), (0,)))
        eqs = [cv == mspl for cv in cvals]
        gs = [jnp.where(eqs[j], cidxs[j], BIG) for j in range(8)]
        g01 = jnp.minimum(jnp.minimum(gs[0], gs[1]), jnp.minimum(gs[2], gs[3]))
        g23 = jnp.minimum(jnp.minimum(gs[4], gs[5]), jnp.minimum(gs[6], gs[7]))
        gspl = _spl_i(lax.reduce_min(jnp.minimum(g01, g23), (0,)))
        sel = iot == k
        outv = jnp.where(sel, mspl, outv)
        outg = jnp.where(sel, gspl, outg)
        for j in range(8):
            rm = eqs[j] & (cidxs[j] == gspl)
            cvals[j] = jnp.where(rm, NEG_INF, cvals[j])

    voc = outg & jnp.int32(VV - 1)
    beam = lax.shift_right_logical(outg, 15)
    ended_g = _dg(ievv, beam)
    newend = jnp.where(voc == END_TOKEN, 1, ended_g)

    smf[...] = outv
    pltpu.sync_copy(smf.at[pl.ds(0, EE)], vals_hbm.at[pl.ds(b * EE, EE)])
    smi[...] = voc
    pltpu.sync_copy(smi.at[pl.ds(0, EE)], voc_hbm.at[pl.ds(b * EE, EE)])
    smi[...] = newend
    pltpu.sync_copy(smi.at[pl.ds(0, EE)], ended_hbm.at[pl.ds(b * EE, EE)])
    smi[...] = beam
    pltpu.sync_copy(smi.at[pl.ds(0, EE)], beam_hbm.at[pl.ds(b * EE, EE)])

    beam2 = _dg(beam, iot & 7)
    off0 = beam2 + jnp.where(iot >= 8, jnp.int32(EE), jnp.int32(0))

    def lp(i, _):
        x = outs_v[pl.ds(i * NLANE, NLANE)]
        outnew_v[pl.ds(i * NLANE, NLANE)] = _dg(x, off0)
        return 0

    lax.fori_loop(0, (LL * EE) // NLANE, lp, 0)
    outnew_v[pl.ds(LL * EE, NLANE)] = voc
    pltpu.sync_copy(outnew_v.at[pl.ds(0, (LL + 1) * EE)],
                    outsnew_hbm.at[pl.ds(b * (LL + 1) * EE, (LL + 1) * EE)])


def kernel(cur_proba, proba, outs, is_ended):
    _, _, V = cur_proba.shape
    L, B, E = outs.shape
    cp = cur_proba.reshape(-1)
    pr = proba.reshape(-1)
    ie = is_ended.astype(jnp.int32).reshape(-1)
    outs_t = outs.transpose(1, 0, 2).reshape(-1)

    mesh = plsc.VectorSubcoreMesh(core_axis_name="c", subcore_axis_name="s",
                                  num_cores=2, num_subcores=16)
    run = functools.partial(
        pl.kernel,
        mesh=mesh,
        compiler_params=pltpu.CompilerParams(needs_layout_passes=False),
        out_type=(
            jax.ShapeDtypeStruct((B * E,), jnp.float32),
            jax.ShapeDtypeStruct((B * E,), jnp.int32),
            jax.ShapeDtypeStruct((B * E,), jnp.int32),
            jax.ShapeDtypeStruct((B * E,), jnp.int32),
            jax.ShapeDtypeStruct((B * (L + 1) * E,), jnp.int32),
        ),
        scratch_types=[
            pltpu.VMEM((CHUNK,), jnp.float32),
            pltpu.VMEM((CHUNK,), jnp.float32),
            pltpu.VMEM((NLANE,), jnp.float32),
            pltpu.VMEM((NLANE,), jnp.int32),
            pltpu.VMEM((E * NLANE,), jnp.float32),
            pltpu.VMEM((L * E,), jnp.int32),
            pltpu.VMEM(((L + 1) * E + 8,), jnp.int32),
            pltpu.VMEM((NLANE,), jnp.float32),
            pltpu.VMEM((NLANE,), jnp.int32),
            pltpu.SemaphoreType.DMA,
            pltpu.SemaphoreType.DMA,
            pltpu.SemaphoreType.DMA,
        ],
    )(_sc_entry)

    vals, voc, beam, ended, outsnew = run(cp, pr, ie, outs_t)

    cur_input = voc.reshape(B * E, 1)
    proba_new = vals.reshape(B, E)
    outs_new = outsnew.reshape(B, L + 1, E).transpose(1, 0, 2)
    is_ended_new = ended.reshape(B, E).astype(bool)
    topk_beam = beam.reshape(B, E)
    return (cur_input, proba_new, outs_new, is_ended_new, topk_beam)


# speedup vs baseline: 3.6338x; 1.0568x over previous
import functools
import jax
import jax.numpy as jnp
from jax import lax
from jax.experimental import pallas as pl
from jax.experimental.pallas import tpu as pltpu
from jax.experimental.pallas import tpu_sc as plsc

END_TOKEN = 2
NEG_INF = float("-inf")
POS_INF = float("inf")
BB, EE, VV, LL = 32, 8, 32768, 128
CHUNK = 16384
SAMP = 8192
GRP = 256
NLANE = 16


def _spl_f(x):
    return lax.broadcast_in_dim(x, (NLANE,), ())


def _spl_i(x):
    return lax.broadcast_in_dim(x, (NLANE,), ())


def _iota():
    return lax.broadcasted_iota(jnp.int32, (NLANE,), 0)


def _dg(x, i):
    dn = lax.GatherDimensionNumbers(offset_dims=(), collapsed_slice_dims=(0,),
                                    start_index_map=(0,))
    return lax.gather(x, lax.broadcast_in_dim(i, (NLANE, 1), (0,)), dn,
                      slice_sizes=(1,), mode=lax.GatherScatterMode.PROMISE_IN_BOUNDS)


def _bubble(cands, v, g):
    vals = list(cands[:8])
    idxs = list(cands[8:])
    for j in range(8):
        gt = v > vals[j]
        nv = jnp.where(gt, v, vals[j])
        ng = jnp.where(gt, g, idxs[j])
        v = jnp.where(gt, vals[j], v)
        g = jnp.where(gt, idxs[j], g)
        vals[j], idxs[j] = nv, ng
    return tuple(vals) + tuple(idxs)


def _sc_entry(cp_hbm, pr_hbm, ie_hbm, outs_hbm,
              vals_hbm, voc_hbm, beam_hbm, ended_hbm, outsnew_hbm,
              buf0, buf1, pv, iev, lmv, outs_v, outnew_v, smf, smi,
              sem0, sem1, sem2):
    ncores = 2
    b = lax.axis_index("s") * ncores + lax.axis_index("c")
    row0 = b * (EE * VV)
    nchunk = (EE * VV) // CHUNK

    def _start(off, bufx, semx):
        pltpu.async_copy(cp_hbm.at[pl.ds(pl.multiple_of(off, 8), CHUNK)],
                         bufx, semx)

    def _wait(bufx, semx):
        pltpu.make_async_copy(cp_hbm.at[pl.ds(0, CHUNK)], bufx, semx).wait()

    def _nxt(g):
        return jnp.where(g + 1 >= nchunk, row0, row0 + (g + 1) * CHUNK)

    pv[...] = jnp.zeros((NLANE,), jnp.float32)
    iev[...] = jnp.zeros((NLANE,), jnp.int32)
    iot = _iota()

    def _start_s(e, bufx, semx):
        off = pl.multiple_of(row0 + e * VV, 8)
        pltpu.async_copy(cp_hbm.at[pl.ds(off, SAMP)],
                         bufx.at[pl.ds(0, SAMP)], semx)

    def _wait_s(bufx, semx):
        pltpu.make_async_copy(cp_hbm.at[pl.ds(0, SAMP)],
                              bufx.at[pl.ds(0, SAMP)], semx).wait()

    _start_s(0, buf0, sem0)
    cp_pr = pltpu.make_async_copy(pr_hbm.at[pl.ds(b * EE, EE)],
                                  pv.at[pl.ds(0, EE)], sem2)
    cp_ie = pltpu.make_async_copy(ie_hbm.at[pl.ds(b * EE, EE)],
                                  iev.at[pl.ds(0, EE)], sem2)
    cp_outs = pltpu.make_async_copy(
        outs_hbm.at[pl.ds(b * (LL * EE), LL * EE)], outs_v, sem2)
    cp_pr.start()
    cp_ie.start()
    cp_outs.start()

    def row_pair_a(ep, _):
        def samp_a(e, bufc, semc, bufn, semn):
            _start_s(jnp.minimum(e + 1, EE - 1), bufn, semn)
            _wait_s(bufc, semc)

            def grp_a(i, accs):
                base = i * GRP
                return tuple(
                    jnp.maximum(accs[k], bufc[pl.ds(base + k * NLANE, NLANE)])
                    for k in range(16))

            accs = tuple(jnp.full((NLANE,), NEG_INF, jnp.float32)
                         for _ in range(16))
            accs = lax.fori_loop(0, SAMP // GRP, grp_a, accs)
            m = list(accs)
            while len(m) > 1:
                m = [jnp.maximum(m[2 * i], m[2 * i + 1])
                     for i in range(len(m) // 2)]
            lmv[pl.ds(e * NLANE, NLANE)] = m[0]

        samp_a(ep * 2, buf0, sem0, buf1, sem1)
        samp_a(ep * 2 + 1, buf1, sem1, buf0, sem0)
        return 0

    lax.fori_loop(0, EE // 2, row_pair_a, 0)
    _wait_s(buf0, sem0)
    _start(row0, buf0, sem0)
    cp_pr.wait()
    cp_ie.wait()
    cp_outs.wait()
    pvv = pv[...]
    ievv = iev[...]

    cvs = []
    for e in range(8):
        idx_e = _spl_i(jnp.int32(e))
        p_spl = _dg(pvv, idx_e)
        end_spl = _dg(ievv, idx_e) > 0
        lane0 = iot == 0
        single = jnp.where(lane0, p_spl, NEG_INF)
        cvs.append(jnp.where(end_spl, single,
                             lmv[pl.ds(e * NLANE, NLANE)] + p_spl))
    tval = None
    for _ in range(8):
        m01 = jnp.maximum(jnp.maximum(cvs[0], cvs[1]),
                          jnp.maximum(cvs[2], cvs[3]))
        m23 = jnp.maximum(jnp.maximum(cvs[4], cvs[5]),
                          jnp.maximum(cvs[6], cvs[7]))
        mscal = lax.reduce_max(jnp.maximum(m01, m23), (0,))
        mspl = _spl_f(mscal)
        cvs = [jnp.where(cv == mspl, NEG_INF, cv) for cv in cvs]
        tval = mspl

    lane0 = iot == 0
    lt8 = iot < 8
    endvec = jnp.where((iev[...] > 0) & lt8, pv[...], NEG_INF)
    endidx = jnp.where(lt8, iot * VV + END_TOKEN, 0)
    cands = ((endvec,) + tuple(jnp.full((NLANE,), NEG_INF, jnp.float32)
                               for _ in range(7))
             + (endidx,) + tuple(jnp.zeros((NLANE,), jnp.int32)
                                 for _ in range(7)))

    def row_b(e, cands):
        idx_e = _spl_i(e)
        p_spl = _dg(pvv, idx_e)
        end_spl = _dg(ievv, idx_e) > 0
        margin = (jnp.abs(tval) + jnp.abs(p_spl) + 1.0) * 1e-5
        trow = jnp.where(end_spl, POS_INF, (tval - p_spl) - margin)
        gb = _spl_i(e * VV) + iot

        def chunk_b(c, bufc, semc, bufn, semn, cands):
            g = e * (VV // CHUNK) + c
            _start(_nxt(g), bufn, semn)
            _wait(bufc, semc)
            gchunk = gb + _spl_i(c * CHUNK)

            def grp_b(i, cands):
                base = i * GRP
                xs = [bufc[pl.ds(base + k * NLANE, NLANE)]
                      for k in range(16)]
                m = list(xs)
                while len(m) > 1:
                    m = [jnp.maximum(m[2 * i2], m[2 * i2 + 1])
                         for i2 in range(len(m) // 2)]
                hit = jnp.any(m[0] >= trow)

                def slow(cands):
                    gk = gchunk + _spl_i(base)
                    for k in range(16):
                        cands = _bubble(cands, xs[k] + p_spl,
                                        gk + _spl_i(k * NLANE))
                    return cands

                return lax.cond(hit, slow, lambda c: c, cands)

            return lax.fori_loop(0, CHUNK // GRP, grp_b, cands)

        cands = chunk_b(0, buf0, sem0, buf1, sem1, cands)
        return chunk_b(1, buf1, sem1, buf0, sem0, cands)

    cands = lax.fori_loop(0, EE, row_b, cands)
    _wait(buf0, sem0)
    cvals = list(cands[:8])
    cidxs = list(cands[8:])

    BIG = jnp.int32(2**30)
    outv = jnp.zeros((NLANE,), jnp.float32)
    outg = jnp.zeros((NLANE,), jnp.int32)
    for k in range(8):
        m01 = jnp.maximum(jnp.maximum(cvals[0], cvals[1]),
                          jnp.maximum(cvals[2], cvals[3]))
        m23 = jnp.maximum(jnp.maximum(cvals[4], cvals[5]),
                          jnp.maximum(cvals[6], cvals[7]))
        mspl = _spl_f(lax.reduce_max(jnp.maximum(m01, m23), (0,)))
        eqs = [cv == mspl for cv in cvals]
        gs = [jnp.where(eqs[j], cidxs[j], BIG) for j in range(8)]
        g01 = jnp.minimum(jnp.minimum(gs[0], gs[1]), jnp.minimum(gs[2], gs[3]))
        g23 = jnp.minimum(jnp.minimum(gs[4], gs[5]), jnp.minimum(gs[6], gs[7]))
        gspl = _spl_i(lax.reduce_min(jnp.minimum(g01, g23), (0,)))
        sel = iot == k
        outv = jnp.where(sel, mspl, outv)
        outg = jnp.where(sel, gspl, outg)
        for j in range(8):
            rm = eqs[j] & (cidxs[j] == gspl)
            cvals[j] = jnp.where(rm, NEG_INF, cvals[j])

    voc = outg & jnp.int32(VV - 1)
    beam = lax.shift_right_logical(outg, 15)
    ended_g = _dg(ievv, beam)
    newend = jnp.where(voc == END_TOKEN, 1, ended_g)

    smf[...] = outv
    pltpu.sync_copy(smf.at[pl.ds(0, EE)], vals_hbm.at[pl.ds(b * EE, EE)])
    smi[...] = voc
    pltpu.sync_copy(smi.at[pl.ds(0, EE)], voc_hbm.at[pl.ds(b * EE, EE)])
    smi[...] = newend
    pltpu.sync_copy(smi.at[pl.ds(0, EE)], ended_hbm.at[pl.ds(b * EE, EE)])
    smi[...] = beam
    pltpu.sync_copy(smi.at[pl.ds(0, EE)], beam_hbm.at[pl.ds(b * EE, EE)])

    beam2 = _dg(beam, iot & 7)
    off0 = beam2 + jnp.where(iot >= 8, jnp.int32(EE), jnp.int32(0))

    def lp(i, _):
        x = outs_v[pl.ds(i * NLANE, NLANE)]
        outnew_v[pl.ds(i * NLANE, NLANE)] = _dg(x, off0)
        return 0

    lax.fori_loop(0, (LL * EE) // NLANE, lp, 0)
    outnew_v[pl.ds(LL * EE, NLANE)] = voc
    pltpu.sync_copy(outnew_v.at[pl.ds(0, (LL + 1) * EE)],
                    outsnew_hbm.at[pl.ds(b * (LL + 1) * EE, (LL + 1) * EE)])


def kernel(cur_proba, proba, outs, is_ended):
    _, _, V = cur_proba.shape
    L, B, E = outs.shape
    cp = cur_proba.reshape(-1)
    pr = proba.reshape(-1)
    ie = is_ended.astype(jnp.int32).reshape(-1)
    outs_t = outs.transpose(1, 0, 2).reshape(-1)

    mesh = plsc.VectorSubcoreMesh(core_axis_name="c", subcore_axis_name="s",
                                  num_cores=2, num_subcores=16)
    run = functools.partial(
        pl.kernel,
        mesh=mesh,
        compiler_params=pltpu.CompilerParams(needs_layout_passes=False),
        out_type=(
            jax.ShapeDtypeStruct((B * E,), jnp.float32),
            jax.ShapeDtypeStruct((B * E,), jnp.int32),
            jax.ShapeDtypeStruct((B * E,), jnp.int32),
            jax.ShapeDtypeStruct((B * E,), jnp.int32),
            jax.ShapeDtypeStruct((B * (L + 1) * E,), jnp.int32),
        ),
        scratch_types=[
            pltpu.VMEM((CHUNK,), jnp.float32),
            pltpu.VMEM((CHUNK,), jnp.float32),
            pltpu.VMEM((NLANE,), jnp.float32),
            pltpu.VMEM((NLANE,), jnp.int32),
            pltpu.VMEM((E * NLANE,), jnp.float32),
            pltpu.VMEM((L * E,), jnp.int32),
            pltpu.VMEM(((L + 1) * E + 8,), jnp.int32),
            pltpu.VMEM((NLANE,), jnp.float32),
            pltpu.VMEM((NLANE,), jnp.int32),
            pltpu.SemaphoreType.DMA,
            pltpu.SemaphoreType.DMA,
            pltpu.SemaphoreType.DMA,
        ],
    )(_sc_entry)

    vals, voc, beam, ended, outsnew = run(cp, pr, ie, outs_t)

    cur_input = voc.reshape(B * E, 1)
    proba_new = vals.reshape(B, E)
    outs_new = outsnew.reshape(B, L + 1, E).transpose(1, 0, 2)
    is_ended_new = ended.reshape(B, E).astype(bool)
    topk_beam = beam.reshape(B, E)
    return (cur_input, proba_new, outs_new, is_ended_new, topk_beam)
